# baseline (speedup 1.0000x reference)
import functools

import jax
import jax.numpy as jnp
import numpy as np
from jax.experimental import pallas as pl
from jax.experimental.pallas import tpu as pltpu


@functools.lru_cache(maxsize=None)
def _bilinear_matrix(dst, src):
    m = np.zeros((dst, src), np.float64)
    scale = src / dst
    for i in range(dst):
        c = (i + 0.5) * scale - 0.5
        lo = int(np.floor(c))
        f = c - lo
        m[i, min(max(lo, 0), src - 1)] += 1.0 - f
        m[i, min(max(lo + 1, 0), src - 1)] += f
    return jnp.asarray(m, jnp.float32)


def _loss_body(o_ref, t_ref, h_ref, out_ref, acc_ref, w_ref):
    i = pl.program_id(0)

    @pl.when(i == 0)
    def _init():
        acc_ref[...] = jnp.zeros_like(acc_ref)
        h = h_ref[...]
        m = jnp.max(h, axis=0, keepdims=True)
        e = jnp.exp(h - m)
        d = jnp.sum(e, axis=0, keepdims=True)
        w_ref[...] = e * pl.reciprocal(d, approx=False)

    hw = w_ref.shape[0]
    cpb = o_ref.shape[0] // hw
    w = w_ref[...]
    d = jnp.abs(o_ref[...] - t_ref[...]).reshape(cpb, hw, w_ref.shape[1])
    acc_ref[...] += jnp.sum(jnp.sum(d, axis=0) * w)

    @pl.when(i == pl.num_programs(0) - 1)
    def _final():
        out_ref[...] = acc_ref[...]


def kernel(output, target, heatmap):
    N, C, H, W = output.shape
    HW = H * W

    out_v = jnp.transpose(output, (1, 2, 3, 0)).reshape(C * HW, N)
    tgt_v = jnp.transpose(target, (1, 2, 3, 0)).reshape(C * HW, N)

    hs, ws = heatmap.shape[2], heatmap.shape[3]
    mh = _bilinear_matrix(H, hs)
    mw = _bilinear_matrix(W, ws)
    hm32 = heatmap.reshape(N, hs, ws).astype(jnp.float32)
    t1 = jnp.einsum("hH,nHW->hnW", mh, hm32)
    up = jnp.einsum("hnW,wW->hwn", t1, mw)
    hm_t = up.reshape(HW, N)

    cpb = 2 if C % 2 == 0 else 1

    loss = pl.pallas_call(
        _loss_body,
        out_shape=jax.ShapeDtypeStruct((1, 1), jnp.float32),
        grid=(C // cpb,),
        in_specs=[
            pl.BlockSpec((cpb * HW, N), lambda i: (i, 0)),
            pl.BlockSpec((cpb * HW, N), lambda i: (i, 0)),
            pl.BlockSpec((HW, N), lambda i: (0, 0)),
        ],
        out_specs=pl.BlockSpec((1, 1), lambda i: (0, 0)),
        scratch_shapes=[
            pltpu.VMEM((1, 1), jnp.float32),
            pltpu.VMEM((HW, N), jnp.float32),
        ],
        compiler_params=pltpu.CompilerParams(
            dimension_semantics=("arbitrary",)),
    )(out_v, tgt_v, hm_t)
    return loss[0, 0]


# speedup vs baseline: 1.0050x; 1.0050x over previous
import functools

import jax
import jax.numpy as jnp
import numpy as np
from jax.experimental import pallas as pl
from jax.experimental.pallas import tpu as pltpu


@functools.lru_cache(maxsize=None)
def _bilinear_matrix(dst, src):
    m = np.zeros((dst, src), np.float64)
    scale = src / dst
    for i in range(dst):
        c = (i + 0.5) * scale - 0.5
        lo = int(np.floor(c))
        f = c - lo
        m[i, min(max(lo, 0), src - 1)] += 1.0 - f
        m[i, min(max(lo + 1, 0), src - 1)] += f
    return jnp.asarray(m, jnp.float32)


def _loss_body(o_ref, t_ref, h_ref, out_ref, acc_ref, w_ref):
    i = pl.program_id(0)

    @pl.when(i == 0)
    def _init():
        acc_ref[...] = jnp.zeros_like(acc_ref)
        h = h_ref[...]
        m = jnp.max(h, axis=0, keepdims=True)
        e = jnp.exp(h - m)
        d = jnp.sum(e, axis=0, keepdims=True)
        w_ref[...] = e * pl.reciprocal(d, approx=False)

    o = o_ref[...]
    t = t_ref[...]
    acc_ref[...] += jnp.sum(jnp.abs(o - t) * w_ref[...])

    @pl.when(i == pl.num_programs(0) - 1)
    def _final():
        out_ref[...] = acc_ref[...]


def kernel(output, target, heatmap):
    N, C, H, W = output.shape
    HW = H * W

    out_v = jnp.transpose(output, (1, 2, 3, 0)).reshape(C * HW, N)
    tgt_v = jnp.transpose(target, (1, 2, 3, 0)).reshape(C * HW, N)

    hs, ws = heatmap.shape[2], heatmap.shape[3]
    mh = _bilinear_matrix(H, hs)
    mw = _bilinear_matrix(W, ws)
    hm32 = heatmap.reshape(N, hs, ws).astype(jnp.float32)
    t1 = jnp.einsum("hH,nHW->hnW", mh, hm32)
    up = jnp.einsum("hnW,wW->hwn", t1, mw)
    hm_t = up.reshape(HW, N)

    loss = pl.pallas_call(
        _loss_body,
        out_shape=jax.ShapeDtypeStruct((1, 1), jnp.float32),
        grid=(C,),
        in_specs=[
            pl.BlockSpec((HW, N), lambda i: (i, 0)),
            pl.BlockSpec((HW, N), lambda i: (i, 0)),
            pl.BlockSpec((HW, N), lambda i: (0, 0)),
        ],
        out_specs=pl.BlockSpec((1, 1), lambda i: (0, 0)),
        scratch_shapes=[
            pltpu.VMEM((1, 1), jnp.float32),
            pltpu.VMEM((HW, N), jnp.float32),
        ],
        compiler_params=pltpu.CompilerParams(
            dimension_semantics=("arbitrary",)),
    )(out_v, tgt_v, hm_t)
    return loss[0, 0]
